# baseline (speedup 1.0000x reference)
import functools

import jax
import jax.numpy as jnp
from jax import lax
from jax.experimental import pallas as pl
from jax.experimental.pallas import tpu as pltpu
from jax.experimental.pallas import tpu_sc as plsc

_N = 10000
_E = 320000
_EPS = 1e-5

_NC = 2
_NS = 16
_NW = _NC * _NS
_EPT = _E // _NW
_KB = 80
_NCHUNK = _EPT // _KB
_HN = 5120
_HR = _HN // 2
_HTRASH = 8
_RZT = _HR // _NS
_EPS2 = _E // _NS
_NCHUNK2 = _EPS2 // _KB

_BE = 2000
_NB = _E // _BE

_mesh = plsc.VectorSubcoreMesh(core_axis_name="c", subcore_axis_name="s")


_NSETS = 5


@functools.partial(
    pl.kernel,
    out_type=jax.ShapeDtypeStruct((_E, 128), jnp.float32),
    mesh=_mesh,
    scratch_types=(
        [pltpu.VMEM((_KB,), jnp.int32) for _ in range(_NSETS)]
        + [pltpu.VMEM((_KB,), jnp.int32) for _ in range(_NSETS)]
        + [pltpu.VMEM((_KB, 128), jnp.float32) for _ in range(_NSETS)]
        + [pltpu.VMEM((_KB, 128), jnp.float32) for _ in range(_NSETS)]
        + [pltpu.SemaphoreType.DMA for _ in range(_NSETS)]
        + [pltpu.SemaphoreType.DMA for _ in range(_NSETS)]
    ),
)
def _sc_gather_add(src_hbm, dst_hbm, hs_hbm, hd_hbm, g_hbm, *scr):
    sidx = scr[0:_NSETS]
    didx = scr[_NSETS:2 * _NSETS]
    abuf = scr[2 * _NSETS:3 * _NSETS]
    bbuf = scr[3 * _NSETS:4 * _NSETS]
    gsem = scr[4 * _NSETS:5 * _NSETS]
    wsem = scr[5 * _NSETS:6 * _NSETS]
    wid = lax.axis_index("s") * _NC + lax.axis_index("c")
    base0 = wid * _EPT

    def group(gi, carry):
        gbase = base0 + gi * (_NSETS * _KB)
        cps = []
        for b in range(_NSETS):
            base = gbase + b * _KB
            pltpu.sync_copy(src_hbm.at[pl.ds(base, _KB)], sidx[b])
            pltpu.sync_copy(dst_hbm.at[pl.ds(base, _KB)], didx[b])
            cpa = pltpu.async_copy(hs_hbm.at[sidx[b]], abuf[b], gsem[b])
            cpb = pltpu.async_copy(hd_hbm.at[didx[b]], bbuf[b], gsem[b])
            cps.append((cpa, cpb))
        wcps = []
        for b in range(_NSETS):
            cps[b][0].wait()
            cps[b][1].wait()

            def row(j, c2, _b=b):
                for c in range(8):
                    sl = pl.ds(c * 16, 16)
                    abuf[_b][j, sl] = abuf[_b][j, sl] + bbuf[_b][j, sl]
                return c2

            lax.fori_loop(0, _KB, row, 0)
            base = gbase + b * _KB
            wcps.append(pltpu.async_copy(abuf[b], g_hbm.at[pl.ds(base, _KB)],
                                         wsem[b]))
        for b in range(_NSETS):
            wcps[b].wait()
        return carry

    lax.fori_loop(0, _NCHUNK // _NSETS, group, 0)


@functools.partial(
    pl.kernel,
    out_type=pltpu.HBM((_NC, _HR, 128), jnp.float32),
    mesh=_mesh,
    scratch_types=(
        [pltpu.VMEM((_KB,), jnp.int32) for _ in range(_NSETS)]
        + [pltpu.VMEM((_KB, 128), jnp.float32) for _ in range(_NSETS)]
        + [pltpu.VMEM((_RZT + _HTRASH, 128), jnp.float32)]
        + [pltpu.VMEM_SHARED((_HR + _HTRASH, 128), jnp.float32)]
        + [pltpu.SemaphoreType.DMA for _ in range(_NSETS)]
        + [pltpu.SemaphoreType.DMA for _ in range(_NSETS)]
    ),
)
def _sc_scatter_add(dst_hbm, msg_hbm, out_hbm, *scr):
    didx = scr[0:_NSETS]
    mbuf = scr[_NSETS:2 * _NSETS]
    zbuf = scr[2 * _NSETS]
    agg_sh = scr[2 * _NSETS + 1]
    msem = scr[2 * _NSETS + 2:3 * _NSETS + 2]
    ssem = scr[3 * _NSETS + 2:4 * _NSETS + 2]
    cid = lax.axis_index("c")
    sid = lax.axis_index("s")
    lo = cid * _HN

    def zrow(j, c2):
        for c in range(8):
            zbuf[j, pl.ds(c * 16, 16)] = jnp.zeros((16,), jnp.float32)
        return c2

    lax.fori_loop(0, _RZT + _HTRASH, zrow, 0)
    rbase = pl.multiple_of(sid * _RZT, 8)

    @pl.when(sid < _NS - 1)
    def _():
        pltpu.sync_copy(zbuf.at[pl.ds(0, _RZT)], agg_sh.at[pl.ds(rbase, _RZT)])

    @pl.when(sid == _NS - 1)
    def _():
        pltpu.sync_copy(zbuf, agg_sh.at[pl.ds(_RZT * (_NS - 1),
                                              _RZT + _HTRASH)])

    plsc.subcore_barrier()

    base0 = sid * _EPS2

    def group(gi, carry):
        gbase = base0 + gi * (_NSETS * _KB)
        cps = []
        for b in range(_NSETS):
            base = gbase + b * _KB
            pltpu.sync_copy(dst_hbm.at[pl.ds(base, _KB)], didx[b])
            cps.append(pltpu.async_copy(msg_hbm.at[pl.ds(base, _KB)],
                                        mbuf[b], msem[b]))
        scps = []
        for b in range(_NSETS):
            cps[b].wait()
            for k in range(_KB // 16):
                d = didx[b][pl.ds(k * 16, 16)]
                lidx = d - lo
                valid = (lidx >= 0) & (lidx < _HN)
                didx[b][pl.ds(k * 16, 16)] = jnp.where(
                    valid, lax.shift_right_logical(lidx, 1), _HR)
            scps.append(pltpu.async_copy(mbuf[b], agg_sh.at[didx[b]],
                                         ssem[b], add=True))
        for b in range(_NSETS):
            scps[b].wait()
        return carry

    lax.fori_loop(0, _NCHUNK2 // _NSETS, group, 0)
    plsc.subcore_barrier()
    pltpu.sync_copy(agg_sh.at[pl.ds(rbase, _RZT)],
                    out_hbm.at[cid, pl.ds(rbase, _RZT)])


def _embed_proj_body(h_ref, w_ref, b_ref, ws_ref, bs_ref, wd_ref, bd_ref,
                     x_ref, hs_ref, hd_ref):
    x = jnp.dot(h_ref[...], w_ref[...],
                preferred_element_type=jnp.float32) + b_ref[...]
    x_ref[...] = x
    hs_ref[...] = jnp.dot(x, ws_ref[...],
                          preferred_element_type=jnp.float32) + bs_ref[...]
    hd_ref[...] = jnp.dot(x, wd_ref[...],
                          preferred_element_type=jnp.float32) + bd_ref[...]


def _ef_body(lh_ref, w1_ref, b1c_ref, w2_ref, b2_ref, o_ref):
    lhb = lh_ref[0]
    step = 8.0 / 39.0
    gamma = 1.0 / (step * step)
    cent = lax.broadcasted_iota(jnp.int32, (40, 1), 0).astype(jnp.float32) * step
    dd = lhb - cent
    rbt = jnp.exp(-gamma * dd * dd)
    h1t = jax.nn.softplus(
        lax.dot_general(w1_ref[...], rbt, (((0,), (0,)), ((), ())),
                        preferred_element_type=jnp.float32) + b1c_ref[...])
    o_ref[...] = lax.dot_general(
        h1t, w2_ref[...], (((0,), (0,)), ((), ())),
        preferred_element_type=jnp.float32) + b2_ref[...]


def _proj_body(x_ref, ws_ref, bs_ref, wd_ref, bd_ref, hs_ref, hd_ref):
    x = x_ref[...]
    hs_ref[...] = jnp.dot(x, ws_ref[...],
                          preferred_element_type=jnp.float32) + bs_ref[...]
    hd_ref[...] = jnp.dot(x, wd_ref[...],
                          preferred_element_type=jnp.float32) + bd_ref[...]


def _statsemit_body(g_ref, ef_ref, w_ref, b_ref, gam_ref, bet_ref,
                    par_ref, o_ref, acc_ref):
    p = pl.program_id(0)
    i = pl.program_id(1)
    m = g_ref[...] + jnp.dot(ef_ref[...], w_ref[...],
                             preferred_element_type=jnp.float32) + b_ref[...]

    @pl.when(p == 0)
    def _():
        s0 = jnp.sum(m, axis=0)
        s1 = jnp.sum(m * m, axis=0)
        blk = jnp.stack([s0, s1], axis=0)

        @pl.when(i == 0)
        def _():
            acc_ref[...] = blk

        @pl.when(i > 0)
        def _():
            acc_ref[...] = acc_ref[...] + blk

    @pl.when(p == 1)
    def _():
        st = acc_ref[...]
        mu = st[0] / _E
        var = st[1] / _E - mu * mu
        scale = gam_ref[0] * lax.rsqrt(var + _EPS)
        shift = bet_ref[0] - mu * scale
        mn = m * scale + shift
        msg = jax.nn.sigmoid(mn[:, :64]) * jax.nn.softplus(mn[:, 64:])
        pe = lax.dot_general(par_ref[0], jnp.ones((1, 1), jnp.float32),
                             (((0,), (0,)), ((), ())),
                             preferred_element_type=jnp.float32)
        o_ref[...] = jnp.concatenate([msg * (1.0 - pe), msg * pe], axis=1)


def _agg_norm(p_ref, g_ref, b_ref):
    agg = jnp.concatenate([p_ref[0], p_ref[1, :_N - _HN]], axis=0)
    mu = jnp.mean(agg, axis=0, keepdims=True)
    var = jnp.mean((agg - mu) ** 2, axis=0, keepdims=True)
    return (agg - mu) * lax.rsqrt(var + _EPS) * g_ref[...] + b_ref[...]


def _upd_proj_body(p_ref, x_ref, g_ref, b_ref, ws_ref, bs_ref, wd_ref,
                   bd_ref, x_out, hs_ref, hd_ref):
    x = jax.nn.softplus(x_ref[...] + _agg_norm(p_ref, g_ref, b_ref))
    x_out[...] = x
    hs_ref[...] = jnp.dot(x, ws_ref[...],
                          preferred_element_type=jnp.float32) + bs_ref[...]
    hd_ref[...] = jnp.dot(x, wd_ref[...],
                          preferred_element_type=jnp.float32) + bd_ref[...]


def _upd_final_body(p_ref, x_ref, g_ref, b_ref, w1_ref, b1_ref, w2_ref,
                    b2_ref, o_ref):
    x = jax.nn.softplus(x_ref[...] + _agg_norm(p_ref, g_ref, b_ref))
    hg = jnp.mean(x, axis=0, keepdims=True)
    t = jax.nn.silu(jnp.dot(hg, w1_ref[...],
                            preferred_element_type=jnp.float32) + b1_ref[...])
    o_ref[...] = jnp.dot(t, w2_ref[...],
                         preferred_element_type=jnp.float32) + b2_ref[...]


_embed_proj_call = pl.pallas_call(
    _embed_proj_body,
    out_shape=[jax.ShapeDtypeStruct((_N, 64), jnp.float32),
               jax.ShapeDtypeStruct((_N, 128), jnp.float32),
               jax.ShapeDtypeStruct((_N, 128), jnp.float32)],
)

_ef_call = pl.pallas_call(
    _ef_body,
    grid=(_NB,),
    in_specs=[
        pl.BlockSpec((1, 1, _BE), lambda i: (i, 0, 0)),
        pl.BlockSpec((40, 64), lambda i: (0, 0)),
        pl.BlockSpec((64, 1), lambda i: (0, 0)),
        pl.BlockSpec((64, 32), lambda i: (0, 0)),
        pl.BlockSpec((1, 32), lambda i: (0, 0)),
    ],
    out_specs=pl.BlockSpec((_BE, 32), lambda i: (i, 0)),
    out_shape=jax.ShapeDtypeStruct((_E, 32), jnp.float32),
)

_statsemit_call = pl.pallas_call(
    _statsemit_body,
    grid=(2, _NB),
    in_specs=[
        pl.BlockSpec((_BE, 128), lambda p, i: (i, 0)),
        pl.BlockSpec((_BE, 32), lambda p, i: (i, 0)),
        pl.BlockSpec((32, 128), lambda p, i: (0, 0)),
        pl.BlockSpec((1, 128), lambda p, i: (0, 0)),
        pl.BlockSpec((1, 128), lambda p, i: (0, 0)),
        pl.BlockSpec((1, 128), lambda p, i: (0, 0)),
        pl.BlockSpec((1, 1, _BE), lambda p, i: (i, 0, 0)),
    ],
    out_specs=pl.BlockSpec((_BE, 128), lambda p, i: (p * i, 0)),
    out_shape=jax.ShapeDtypeStruct((_E, 128), jnp.float32),
    scratch_shapes=[pltpu.VMEM((2, 128), jnp.float32)],
)

_upd_proj_call = pl.pallas_call(
    _upd_proj_body,
    out_shape=[jax.ShapeDtypeStruct((_N, 64), jnp.float32),
               jax.ShapeDtypeStruct((_N, 128), jnp.float32),
               jax.ShapeDtypeStruct((_N, 128), jnp.float32)],
)

_upd_final_call = pl.pallas_call(
    _upd_final_body,
    out_shape=jax.ShapeDtypeStruct((1, 1), jnp.float32),
)


def kernel(edge_index, h, e, lh, W_emb, b_emb, W_r1, b_r1, W_r2, b_r2,
           Wsrc, bsrc, Wdst, bdst, Wedge, bedge,
           bn_m_g, bn_m_b, bn_g, bn_b, W_f1, b_f1, W_f2, b_f2):
    src = edge_index[0]
    dst = edge_index[1]
    dpar = (dst & 1).astype(jnp.float32).reshape(_NB, 1, _BE)

    x, hs, hd = _embed_proj_call(h, W_emb, b_emb.reshape(1, -1),
                                 Wsrc[0], bsrc[0].reshape(1, -1),
                                 Wdst[0], bdst[0].reshape(1, -1))
    ef = _ef_call(lh.reshape(_NB, 1, _BE), W_r1, b_r1.reshape(-1, 1),
                  W_r2, b_r2.reshape(1, -1))

    for l in range(3):
        g = _sc_gather_add(src, dst, hs, hd)
        msg = _statsemit_call(g, ef, Wedge[l], bedge[l].reshape(1, -1),
                              bn_m_g[l].reshape(1, -1),
                              bn_m_b[l].reshape(1, -1), dpar)
        parts = _sc_scatter_add(dst, msg)
        parts = parts.reshape(_NC, _HN, 64)
        if l < 2:
            x, hs, hd = _upd_proj_call(parts, x, bn_g[l].reshape(1, -1),
                                       bn_b[l].reshape(1, -1),
                                       Wsrc[l + 1], bsrc[l + 1].reshape(1, -1),
                                       Wdst[l + 1], bdst[l + 1].reshape(1, -1))
        else:
            out = _upd_final_call(parts, x, bn_g[l].reshape(1, -1),
                                  bn_b[l].reshape(1, -1),
                                  W_f1, b_f1.reshape(1, -1),
                                  W_f2, b_f2.reshape(1, -1))

    return out.reshape(1)


# speedup vs baseline: 1.0451x; 1.0451x over previous
import functools

import jax
import jax.numpy as jnp
from jax import lax
from jax.experimental import pallas as pl
from jax.experimental.pallas import tpu as pltpu
from jax.experimental.pallas import tpu_sc as plsc

_N = 10000
_E = 320000
_EPS = 1e-5

_NC = 2
_NS = 16
_NW = _NC * _NS
_EPT = _E // _NW
_KB = 80
_NCHUNK = _EPT // _KB
_HN = 5120
_HR = _HN // 2
_HTRASH = 8
_RZT = _HR // _NS
_EPS2 = _E // _NS
_NCHUNK2 = _EPS2 // _KB

_BE = 2000
_NB = _E // _BE

_mesh = plsc.VectorSubcoreMesh(core_axis_name="c", subcore_axis_name="s")


_NSETS = 5


@functools.partial(
    pl.kernel,
    out_type=jax.ShapeDtypeStruct((_E, 128), jnp.float32),
    mesh=_mesh,
    scratch_types=(
        [pltpu.VMEM((_KB,), jnp.int32) for _ in range(_NSETS)]
        + [pltpu.VMEM((_KB,), jnp.int32) for _ in range(_NSETS)]
        + [pltpu.VMEM((_KB, 128), jnp.float32) for _ in range(_NSETS)]
        + [pltpu.VMEM((_KB, 128), jnp.float32) for _ in range(_NSETS)]
        + [pltpu.SemaphoreType.DMA for _ in range(_NSETS)]
        + [pltpu.SemaphoreType.DMA for _ in range(_NSETS)]
    ),
)
def _sc_gather_add(src_hbm, dst_hbm, hs_hbm, hd_hbm, g_hbm, *scr):
    sidx = scr[0:_NSETS]
    didx = scr[_NSETS:2 * _NSETS]
    abuf = scr[2 * _NSETS:3 * _NSETS]
    bbuf = scr[3 * _NSETS:4 * _NSETS]
    gsem = scr[4 * _NSETS:5 * _NSETS]
    wsem = scr[5 * _NSETS:6 * _NSETS]
    wid = lax.axis_index("s") * _NC + lax.axis_index("c")
    base0 = wid * _EPT

    def group(gi, carry):
        gbase = base0 + gi * (_NSETS * _KB)
        cps = []
        for b in range(_NSETS):
            base = gbase + b * _KB
            pltpu.sync_copy(src_hbm.at[pl.ds(base, _KB)], sidx[b])
            pltpu.sync_copy(dst_hbm.at[pl.ds(base, _KB)], didx[b])
            cpa = pltpu.async_copy(hs_hbm.at[sidx[b]], abuf[b], gsem[b])
            cpb = pltpu.async_copy(hd_hbm.at[didx[b]], bbuf[b], gsem[b])
            cps.append((cpa, cpb))
        wcps = []
        for b in range(_NSETS):
            cps[b][0].wait()
            cps[b][1].wait()

            def row(j, c2, _b=b):
                for c in range(8):
                    sl = pl.ds(c * 16, 16)
                    abuf[_b][j, sl] = abuf[_b][j, sl] + bbuf[_b][j, sl]
                return c2

            lax.fori_loop(0, _KB, row, 0)
            base = gbase + b * _KB
            wcps.append(pltpu.async_copy(abuf[b], g_hbm.at[pl.ds(base, _KB)],
                                         wsem[b]))
        for b in range(_NSETS):
            wcps[b].wait()
        return carry

    lax.fori_loop(0, _NCHUNK // _NSETS, group, 0)


@functools.partial(
    pl.kernel,
    out_type=pltpu.HBM((_NC, _HR, 128), jnp.float32),
    mesh=_mesh,
    scratch_types=(
        [pltpu.VMEM((_KB,), jnp.int32) for _ in range(_NSETS)]
        + [pltpu.VMEM((_KB, 128), jnp.float32) for _ in range(_NSETS)]
        + [pltpu.VMEM((_RZT + _HTRASH, 128), jnp.float32)]
        + [pltpu.VMEM_SHARED((_HR + _HTRASH, 128), jnp.float32)]
        + [pltpu.SemaphoreType.DMA for _ in range(_NSETS)]
        + [pltpu.SemaphoreType.DMA for _ in range(_NSETS)]
    ),
)
def _sc_scatter_add(dst_hbm, msg_hbm, out_hbm, *scr):
    didx = scr[0:_NSETS]
    mbuf = scr[_NSETS:2 * _NSETS]
    zbuf = scr[2 * _NSETS]
    agg_sh = scr[2 * _NSETS + 1]
    msem = scr[2 * _NSETS + 2:3 * _NSETS + 2]
    ssem = scr[3 * _NSETS + 2:4 * _NSETS + 2]
    cid = lax.axis_index("c")
    sid = lax.axis_index("s")
    lo = cid * _HN

    def zrow(j, c2):
        for c in range(8):
            zbuf[j, pl.ds(c * 16, 16)] = jnp.zeros((16,), jnp.float32)
        return c2

    lax.fori_loop(0, _RZT + _HTRASH, zrow, 0)
    rbase = pl.multiple_of(sid * _RZT, 8)

    @pl.when(sid < _NS - 1)
    def _():
        pltpu.sync_copy(zbuf.at[pl.ds(0, _RZT)], agg_sh.at[pl.ds(rbase, _RZT)])

    @pl.when(sid == _NS - 1)
    def _():
        pltpu.sync_copy(zbuf, agg_sh.at[pl.ds(_RZT * (_NS - 1),
                                              _RZT + _HTRASH)])

    plsc.subcore_barrier()

    base0 = sid * _EPS2

    def group(gi, carry):
        gbase = base0 + gi * (_NSETS * _KB)
        cps = []
        for b in range(_NSETS):
            base = gbase + b * _KB
            pltpu.sync_copy(dst_hbm.at[pl.ds(base, _KB)], didx[b])
            cps.append(pltpu.async_copy(msg_hbm.at[pl.ds(base, _KB)],
                                        mbuf[b], msem[b]))
        scps = []
        for b in range(_NSETS):
            cps[b].wait()
            for k in range(_KB // 16):
                d = didx[b][pl.ds(k * 16, 16)]
                lidx = d - lo
                valid = (lidx >= 0) & (lidx < _HN)
                didx[b][pl.ds(k * 16, 16)] = jnp.where(
                    valid, lax.shift_right_logical(lidx, 1), _HR)
            scps.append(pltpu.async_copy(mbuf[b], agg_sh.at[didx[b]],
                                         ssem[b], add=True))
        for b in range(_NSETS):
            scps[b].wait()
        return carry

    lax.fori_loop(0, _NCHUNK2 // _NSETS, group, 0)
    plsc.subcore_barrier()
    pltpu.sync_copy(agg_sh.at[pl.ds(rbase, _RZT)],
                    out_hbm.at[cid, pl.ds(rbase, _RZT)])


def _embed_proj_body(h_ref, w_ref, b_ref, ws_ref, bs_ref, wd_ref, bd_ref,
                     x_ref, hs_ref, hd_ref):
    x = jnp.dot(h_ref[...], w_ref[...],
                preferred_element_type=jnp.float32) + b_ref[...]
    x_ref[...] = x
    hs_ref[...] = jnp.dot(x, ws_ref[...],
                          preferred_element_type=jnp.float32) + bs_ref[...]
    hd_ref[...] = jnp.dot(x, wd_ref[...],
                          preferred_element_type=jnp.float32) + bd_ref[...]


def _ef_body(lh_ref, w1_ref, b1c_ref, w2_ref, b2_ref, o_ref):
    lhb = lh_ref[0]
    step = 8.0 / 39.0
    gamma = 1.0 / (step * step)
    cent = lax.broadcasted_iota(jnp.int32, (40, 1), 0).astype(jnp.float32) * step
    dd = lhb - cent
    rbt = jnp.exp(-gamma * dd * dd)
    h1t = jax.nn.softplus(
        lax.dot_general(w1_ref[...], rbt, (((0,), (0,)), ((), ())),
                        preferred_element_type=jnp.float32) + b1c_ref[...])
    o_ref[...] = lax.dot_general(
        h1t, w2_ref[...], (((0,), (0,)), ((), ())),
        preferred_element_type=jnp.float32) + b2_ref[...]


def _proj_body(x_ref, ws_ref, bs_ref, wd_ref, bd_ref, hs_ref, hd_ref):
    x = x_ref[...]
    hs_ref[...] = jnp.dot(x, ws_ref[...],
                          preferred_element_type=jnp.float32) + bs_ref[...]
    hd_ref[...] = jnp.dot(x, wd_ref[...],
                          preferred_element_type=jnp.float32) + bd_ref[...]


def _stats_body(g_ref, ef_ref, w_ref, b_ref, o_ref):
    i = pl.program_id(0)
    m = g_ref[...] + jnp.dot(ef_ref[...], w_ref[...],
                             preferred_element_type=jnp.float32) + b_ref[...]
    s0 = jnp.sum(m, axis=0)
    s1 = jnp.sum(m * m, axis=0)
    blk = jnp.stack([s0, s1], axis=0)

    @pl.when(i == 0)
    def _():
        o_ref[...] = blk

    @pl.when(i > 0)
    def _():
        o_ref[...] = o_ref[...] + blk


def _emit_body(g_ref, ef_ref, w_ref, b_ref, st_ref, gam_ref, bet_ref,
               par_ref, o_ref):
    st = st_ref[...]
    mu = st[0] / _E
    var = st[1] / _E - mu * mu
    scale = gam_ref[0] * lax.rsqrt(var + _EPS)
    shift = bet_ref[0] - mu * scale
    m = g_ref[...] + jnp.dot(ef_ref[...], w_ref[...],
                             preferred_element_type=jnp.float32) + b_ref[...]
    mn = m * scale + shift
    msg = jax.nn.sigmoid(mn[:, :64]) * jax.nn.softplus(mn[:, 64:])
    pe = lax.dot_general(par_ref[0], jnp.ones((1, 1), jnp.float32),
                         (((0,), (0,)), ((), ())),
                         preferred_element_type=jnp.float32)
    o_ref[...] = jnp.concatenate([msg * (1.0 - pe), msg * pe], axis=1)


def _agg_norm(p_ref, g_ref, b_ref):
    agg = jnp.concatenate([p_ref[0], p_ref[1, :_N - _HN]], axis=0)
    mu = jnp.mean(agg, axis=0, keepdims=True)
    var = jnp.mean((agg - mu) ** 2, axis=0, keepdims=True)
    return (agg - mu) * lax.rsqrt(var + _EPS) * g_ref[...] + b_ref[...]


def _upd_proj_body(p_ref, x_ref, g_ref, b_ref, ws_ref, bs_ref, wd_ref,
                   bd_ref, x_out, hs_ref, hd_ref):
    x = jax.nn.softplus(x_ref[...] + _agg_norm(p_ref, g_ref, b_ref))
    x_out[...] = x
    hs_ref[...] = jnp.dot(x, ws_ref[...],
                          preferred_element_type=jnp.float32) + bs_ref[...]
    hd_ref[...] = jnp.dot(x, wd_ref[...],
                          preferred_element_type=jnp.float32) + bd_ref[...]


def _upd_final_body(p_ref, x_ref, g_ref, b_ref, w1_ref, b1_ref, w2_ref,
                    b2_ref, o_ref):
    x = jax.nn.softplus(x_ref[...] + _agg_norm(p_ref, g_ref, b_ref))
    hg = jnp.mean(x, axis=0, keepdims=True)
    t = jax.nn.silu(jnp.dot(hg, w1_ref[...],
                            preferred_element_type=jnp.float32) + b1_ref[...])
    o_ref[...] = jnp.dot(t, w2_ref[...],
                         preferred_element_type=jnp.float32) + b2_ref[...]


_embed_proj_call = pl.pallas_call(
    _embed_proj_body,
    out_shape=[jax.ShapeDtypeStruct((_N, 64), jnp.float32),
               jax.ShapeDtypeStruct((_N, 128), jnp.float32),
               jax.ShapeDtypeStruct((_N, 128), jnp.float32)],
)

_ef_call = pl.pallas_call(
    _ef_body,
    grid=(_NB,),
    in_specs=[
        pl.BlockSpec((1, 1, _BE), lambda i: (i, 0, 0)),
        pl.BlockSpec((40, 64), lambda i: (0, 0)),
        pl.BlockSpec((64, 1), lambda i: (0, 0)),
        pl.BlockSpec((64, 32), lambda i: (0, 0)),
        pl.BlockSpec((1, 32), lambda i: (0, 0)),
    ],
    out_specs=pl.BlockSpec((_BE, 32), lambda i: (i, 0)),
    out_shape=jax.ShapeDtypeStruct((_E, 32), jnp.float32),
)

_stats_call = pl.pallas_call(
    _stats_body,
    grid=(_NB,),
    in_specs=[
        pl.BlockSpec((_BE, 128), lambda i: (i, 0)),
        pl.BlockSpec((_BE, 32), lambda i: (i, 0)),
        pl.BlockSpec((32, 128), lambda i: (0, 0)),
        pl.BlockSpec((1, 128), lambda i: (0, 0)),
    ],
    out_specs=pl.BlockSpec((2, 128), lambda i: (0, 0)),
    out_shape=jax.ShapeDtypeStruct((2, 128), jnp.float32),
)

_emit_call = pl.pallas_call(
    _emit_body,
    grid=(_NB,),
    in_specs=[
        pl.BlockSpec((_BE, 128), lambda i: (i, 0)),
        pl.BlockSpec((_BE, 32), lambda i: (i, 0)),
        pl.BlockSpec((32, 128), lambda i: (0, 0)),
        pl.BlockSpec((1, 128), lambda i: (0, 0)),
        pl.BlockSpec((2, 128), lambda i: (0, 0)),
        pl.BlockSpec((1, 128), lambda i: (0, 0)),
        pl.BlockSpec((1, 128), lambda i: (0, 0)),
        pl.BlockSpec((1, 1, _BE), lambda i: (i, 0, 0)),
    ],
    out_specs=pl.BlockSpec((_BE, 128), lambda i: (i, 0)),
    out_shape=jax.ShapeDtypeStruct((_E, 128), jnp.float32),
)

_upd_proj_call = pl.pallas_call(
    _upd_proj_body,
    out_shape=[jax.ShapeDtypeStruct((_N, 64), jnp.float32),
               jax.ShapeDtypeStruct((_N, 128), jnp.float32),
               jax.ShapeDtypeStruct((_N, 128), jnp.float32)],
)

_upd_final_call = pl.pallas_call(
    _upd_final_body,
    out_shape=jax.ShapeDtypeStruct((1, 1), jnp.float32),
)


def kernel(edge_index, h, e, lh, W_emb, b_emb, W_r1, b_r1, W_r2, b_r2,
           Wsrc, bsrc, Wdst, bdst, Wedge, bedge,
           bn_m_g, bn_m_b, bn_g, bn_b, W_f1, b_f1, W_f2, b_f2):
    src = edge_index[0]
    dst = edge_index[1]
    dpar = (dst & 1).astype(jnp.float32).reshape(_NB, 1, _BE)

    x, hs, hd = _embed_proj_call(h, W_emb, b_emb.reshape(1, -1),
                                 Wsrc[0], bsrc[0].reshape(1, -1),
                                 Wdst[0], bdst[0].reshape(1, -1))
    ef = _ef_call(lh.reshape(_NB, 1, _BE), W_r1, b_r1.reshape(-1, 1),
                  W_r2, b_r2.reshape(1, -1))

    for l in range(3):
        g = _sc_gather_add(src, dst, hs, hd)
        st = _stats_call(g, ef, Wedge[l], bedge[l].reshape(1, -1))
        msg = _emit_call(g, ef, Wedge[l], bedge[l].reshape(1, -1), st,
                         bn_m_g[l].reshape(1, -1),
                         bn_m_b[l].reshape(1, -1), dpar)
        parts = _sc_scatter_add(dst, msg)
        parts = parts.reshape(_NC, _HN, 64)
        if l < 2:
            x, hs, hd = _upd_proj_call(parts, x, bn_g[l].reshape(1, -1),
                                       bn_b[l].reshape(1, -1),
                                       Wsrc[l + 1], bsrc[l + 1].reshape(1, -1),
                                       Wdst[l + 1], bdst[l + 1].reshape(1, -1))
        else:
            out = _upd_final_call(parts, x, bn_g[l].reshape(1, -1),
                                  bn_b[l].reshape(1, -1),
                                  W_f1, b_f1.reshape(1, -1),
                                  W_f2, b_f2.reshape(1, -1))

    return out.reshape(1)


# speedup vs baseline: 1.1800x; 1.1291x over previous
import functools

import jax
import jax.numpy as jnp
from jax import lax
from jax.experimental import pallas as pl
from jax.experimental.pallas import tpu as pltpu
from jax.experimental.pallas import tpu_sc as plsc

_N = 10000
_E = 320000
_EPS = 1e-5

_NC = 2
_NS = 16
_NW = _NC * _NS
_EPT = _E // _NW
_KB = 80
_NCHUNK = _EPT // _KB
_HN = 5120
_HR = _HN // 2
_HTRASH = 8
_RZT = _HR // _NS
_EPS2 = _E // _NS
_NCHUNK2 = _EPS2 // _KB

_BE = 2000
_NB = _E // _BE

_mesh = plsc.VectorSubcoreMesh(core_axis_name="c", subcore_axis_name="s")


_NSETS = 5


@functools.partial(
    pl.kernel,
    out_type=jax.ShapeDtypeStruct((_E, 128), jnp.float32),
    mesh=_mesh,
    scratch_types=(
        [pltpu.VMEM((_KB,), jnp.int32) for _ in range(_NSETS)]
        + [pltpu.VMEM((_KB,), jnp.int32) for _ in range(_NSETS)]
        + [pltpu.VMEM((_KB, 128), jnp.float32) for _ in range(_NSETS)]
        + [pltpu.VMEM((_KB, 128), jnp.float32) for _ in range(_NSETS)]
        + [pltpu.SemaphoreType.DMA for _ in range(_NSETS)]
        + [pltpu.SemaphoreType.DMA for _ in range(_NSETS)]
    ),
)
def _sc_gather_add(src_hbm, dst_hbm, hs_hbm, hd_hbm, g_hbm, *scr):
    sidx = scr[0:_NSETS]
    didx = scr[_NSETS:2 * _NSETS]
    abuf = scr[2 * _NSETS:3 * _NSETS]
    bbuf = scr[3 * _NSETS:4 * _NSETS]
    gsem = scr[4 * _NSETS:5 * _NSETS]
    wsem = scr[5 * _NSETS:6 * _NSETS]
    wid = lax.axis_index("s") * _NC + lax.axis_index("c")
    base0 = wid * _EPT

    def group(gi, carry):
        gbase = base0 + gi * (_NSETS * _KB)
        cps = []
        for b in range(_NSETS):
            base = gbase + b * _KB
            pltpu.sync_copy(src_hbm.at[pl.ds(base, _KB)], sidx[b])
            pltpu.sync_copy(dst_hbm.at[pl.ds(base, _KB)], didx[b])
            cpa = pltpu.async_copy(hs_hbm.at[sidx[b]], abuf[b], gsem[b])
            cpb = pltpu.async_copy(hd_hbm.at[didx[b]], bbuf[b], gsem[b])
            cps.append((cpa, cpb))
        wcps = []
        for b in range(_NSETS):
            cps[b][0].wait()
            cps[b][1].wait()

            def row(j, c2, _b=b):
                for c in range(8):
                    sl = pl.ds(c * 16, 16)
                    abuf[_b][j, sl] = abuf[_b][j, sl] + bbuf[_b][j, sl]
                return c2

            lax.fori_loop(0, _KB, row, 0)
            base = gbase + b * _KB
            wcps.append(pltpu.async_copy(abuf[b], g_hbm.at[pl.ds(base, _KB)],
                                         wsem[b]))
        for b in range(_NSETS):
            wcps[b].wait()
        return carry

    lax.fori_loop(0, _NCHUNK // _NSETS, group, 0)


def _make_scatter(ecount):
    ept = ecount // _NS
    nchunk = ept // _KB

    @functools.partial(
        pl.kernel,
        out_type=pltpu.HBM((_NC, _HR, 128), jnp.float32),
        mesh=_mesh,
        scratch_types=(
            [pltpu.VMEM((_KB,), jnp.int32) for _ in range(_NSETS)]
            + [pltpu.VMEM((_KB, 128), jnp.float32) for _ in range(_NSETS)]
            + [pltpu.VMEM((_RZT + _HTRASH, 128), jnp.float32)]
            + [pltpu.VMEM_SHARED((_HR + _HTRASH, 128), jnp.float32)]
            + [pltpu.SemaphoreType.DMA for _ in range(_NSETS)]
            + [pltpu.SemaphoreType.DMA for _ in range(_NSETS)]
        ),
    )
    def _sc_scatter_add(dst_hbm, msg_hbm, out_hbm, *scr):
        didx = scr[0:_NSETS]
        mbuf = scr[_NSETS:2 * _NSETS]
        zbuf = scr[2 * _NSETS]
        agg_sh = scr[2 * _NSETS + 1]
        msem = scr[2 * _NSETS + 2:3 * _NSETS + 2]
        ssem = scr[3 * _NSETS + 2:4 * _NSETS + 2]
        cid = lax.axis_index("c")
        sid = lax.axis_index("s")
        lo = cid * _HN

        def zrow(j, c2):
            for c in range(8):
                zbuf[j, pl.ds(c * 16, 16)] = jnp.zeros((16,), jnp.float32)
            return c2

        lax.fori_loop(0, _RZT + _HTRASH, zrow, 0)
        rbase = pl.multiple_of(sid * _RZT, 8)

        @pl.when(sid < _NS - 1)
        def _():
            pltpu.sync_copy(zbuf.at[pl.ds(0, _RZT)],
                            agg_sh.at[pl.ds(rbase, _RZT)])

        @pl.when(sid == _NS - 1)
        def _():
            pltpu.sync_copy(zbuf, agg_sh.at[pl.ds(_RZT * (_NS - 1),
                                                  _RZT + _HTRASH)])

        plsc.subcore_barrier()

        base0 = sid * ept

        def group(gi, carry):
            gbase = base0 + gi * (_NSETS * _KB)
            cps = []
            for b in range(_NSETS):
                base = gbase + b * _KB
                pltpu.sync_copy(dst_hbm.at[pl.ds(base, _KB)], didx[b])
                cps.append(pltpu.async_copy(msg_hbm.at[pl.ds(base, _KB)],
                                            mbuf[b], msem[b]))
            scps = []
            for b in range(_NSETS):
                cps[b].wait()
                for k in range(_KB // 16):
                    d = didx[b][pl.ds(k * 16, 16)]
                    lidx = d - lo
                    valid = (lidx >= 0) & (lidx < _HN)
                    didx[b][pl.ds(k * 16, 16)] = jnp.where(
                        valid, lax.shift_right_logical(lidx, 1), _HR)
                scps.append(pltpu.async_copy(mbuf[b], agg_sh.at[didx[b]],
                                             ssem[b], add=True))
            for b in range(_NSETS):
                scps[b].wait()
            return carry

        lax.fori_loop(0, nchunk // _NSETS, group, 0)
        plsc.subcore_barrier()
        pltpu.sync_copy(agg_sh.at[pl.ds(rbase, _RZT)],
                        out_hbm.at[cid, pl.ds(rbase, _RZT)])

    return _sc_scatter_add


_sc_scatter_half = _make_scatter(_E // 2)


def _embed_proj_body(h_ref, w_ref, b_ref, ws_ref, bs_ref, wd_ref, bd_ref,
                     x_ref, hs_ref, hd_ref):
    x = jnp.dot(h_ref[...], w_ref[...],
                preferred_element_type=jnp.float32) + b_ref[...]
    x_ref[...] = x
    hs_ref[...] = jnp.dot(x, ws_ref[...],
                          preferred_element_type=jnp.float32) + bs_ref[...]
    hd_ref[...] = jnp.dot(x, wd_ref[...],
                          preferred_element_type=jnp.float32) + bd_ref[...]


def _ef_body(lh_ref, w1_ref, b1c_ref, w2_ref, b2_ref, o_ref):
    lhb = lh_ref[0]
    step = 8.0 / 39.0
    gamma = 1.0 / (step * step)
    cent = lax.broadcasted_iota(jnp.int32, (40, 1), 0).astype(jnp.float32) * step
    dd = lhb - cent
    rbt = jnp.exp(-gamma * dd * dd)
    h1t = jax.nn.softplus(
        lax.dot_general(w1_ref[...], rbt, (((0,), (0,)), ((), ())),
                        preferred_element_type=jnp.float32) + b1c_ref[...])
    o_ref[...] = lax.dot_general(
        h1t, w2_ref[...], (((0,), (0,)), ((), ())),
        preferred_element_type=jnp.float32) + b2_ref[...]


def _proj_body(x_ref, ws_ref, bs_ref, wd_ref, bd_ref, hs_ref, hd_ref):
    x = x_ref[...]
    hs_ref[...] = jnp.dot(x, ws_ref[...],
                          preferred_element_type=jnp.float32) + bs_ref[...]
    hd_ref[...] = jnp.dot(x, wd_ref[...],
                          preferred_element_type=jnp.float32) + bd_ref[...]


def _stats_body(g_ref, ef_ref, w_ref, b_ref, o_ref):
    i = pl.program_id(0)
    m = g_ref[...] + jnp.dot(ef_ref[...], w_ref[...],
                             preferred_element_type=jnp.float32) + b_ref[...]
    s0 = jnp.sum(m, axis=0)
    s1 = jnp.sum(m * m, axis=0)
    blk = jnp.stack([s0, s1], axis=0)

    @pl.when(i == 0)
    def _():
        o_ref[...] = blk

    @pl.when(i > 0)
    def _():
        o_ref[...] = o_ref[...] + blk


def _emit_body(g_ref, ef_ref, w_ref, b_ref, st_ref, gam_ref, bet_ref,
               par_ref, o_ref):
    st = st_ref[...]
    mu = st[0] / _E
    var = st[1] / _E - mu * mu
    scale = gam_ref[0] * lax.rsqrt(var + _EPS)
    shift = bet_ref[0] - mu * scale
    m = g_ref[...] + jnp.dot(ef_ref[...], w_ref[...],
                             preferred_element_type=jnp.float32) + b_ref[...]
    mn = m * scale + shift
    msg = jax.nn.sigmoid(mn[:, :64]) * jax.nn.softplus(mn[:, 64:])
    pe = lax.dot_general(par_ref[0], jnp.ones((1, 1), jnp.float32),
                         (((0,), (0,)), ((), ())),
                         preferred_element_type=jnp.float32)
    o_ref[...] = jnp.concatenate([msg * (1.0 - pe), msg * pe], axis=1)


def _agg_norm(p0_ref, p1_ref, g_ref, b_ref):
    agg = jnp.concatenate(
        [p0_ref[0] + p1_ref[0], (p0_ref[1] + p1_ref[1])[:_N - _HN]], axis=0)
    mu = jnp.mean(agg, axis=0, keepdims=True)
    var = jnp.mean((agg - mu) ** 2, axis=0, keepdims=True)
    return (agg - mu) * lax.rsqrt(var + _EPS) * g_ref[...] + b_ref[...]


def _upd_proj_body(p0_ref, p1_ref, x_ref, g_ref, b_ref, ws_ref, bs_ref,
                   wd_ref, bd_ref, x_out, hs_ref, hd_ref):
    x = jax.nn.softplus(x_ref[...] + _agg_norm(p0_ref, p1_ref, g_ref, b_ref))
    x_out[...] = x
    hs_ref[...] = jnp.dot(x, ws_ref[...],
                          preferred_element_type=jnp.float32) + bs_ref[...]
    hd_ref[...] = jnp.dot(x, wd_ref[...],
                          preferred_element_type=jnp.float32) + bd_ref[...]


def _upd_final_body(p0_ref, p1_ref, x_ref, g_ref, b_ref, w1_ref, b1_ref,
                    w2_ref, b2_ref, o_ref):
    x = jax.nn.softplus(x_ref[...] + _agg_norm(p0_ref, p1_ref, g_ref, b_ref))
    hg = jnp.mean(x, axis=0, keepdims=True)
    t = jax.nn.silu(jnp.dot(hg, w1_ref[...],
                            preferred_element_type=jnp.float32) + b1_ref[...])
    o_ref[...] = jnp.dot(t, w2_ref[...],
                         preferred_element_type=jnp.float32) + b2_ref[...]


_embed_proj_call = pl.pallas_call(
    _embed_proj_body,
    out_shape=[jax.ShapeDtypeStruct((_N, 64), jnp.float32),
               jax.ShapeDtypeStruct((_N, 128), jnp.float32),
               jax.ShapeDtypeStruct((_N, 128), jnp.float32)],
)

_ef_call = pl.pallas_call(
    _ef_body,
    grid=(_NB,),
    in_specs=[
        pl.BlockSpec((1, 1, _BE), lambda i: (i, 0, 0)),
        pl.BlockSpec((40, 64), lambda i: (0, 0)),
        pl.BlockSpec((64, 1), lambda i: (0, 0)),
        pl.BlockSpec((64, 32), lambda i: (0, 0)),
        pl.BlockSpec((1, 32), lambda i: (0, 0)),
    ],
    out_specs=pl.BlockSpec((_BE, 32), lambda i: (i, 0)),
    out_shape=jax.ShapeDtypeStruct((_E, 32), jnp.float32),
)

_stats_call = pl.pallas_call(
    _stats_body,
    grid=(_NB,),
    in_specs=[
        pl.BlockSpec((_BE, 128), lambda i: (i, 0)),
        pl.BlockSpec((_BE, 32), lambda i: (i, 0)),
        pl.BlockSpec((32, 128), lambda i: (0, 0)),
        pl.BlockSpec((1, 128), lambda i: (0, 0)),
    ],
    out_specs=pl.BlockSpec((2, 128), lambda i: (0, 0)),
    out_shape=jax.ShapeDtypeStruct((2, 128), jnp.float32),
)

def _make_emit(h):
    off = h * (_NB // 2)
    return pl.pallas_call(
        _emit_body,
        grid=(_NB // 2,),
        in_specs=[
            pl.BlockSpec((_BE, 128), lambda i: (i + off, 0)),
            pl.BlockSpec((_BE, 32), lambda i: (i + off, 0)),
            pl.BlockSpec((32, 128), lambda i: (0, 0)),
            pl.BlockSpec((1, 128), lambda i: (0, 0)),
            pl.BlockSpec((2, 128), lambda i: (0, 0)),
            pl.BlockSpec((1, 128), lambda i: (0, 0)),
            pl.BlockSpec((1, 128), lambda i: (0, 0)),
            pl.BlockSpec((1, 1, _BE), lambda i: (i + off, 0, 0)),
        ],
        out_specs=pl.BlockSpec((_BE, 128), lambda i: (i, 0)),
        out_shape=jax.ShapeDtypeStruct((_E // 2, 128), jnp.float32),
    )


_emit_half = [_make_emit(0), _make_emit(1)]

_upd_proj_call = pl.pallas_call(
    _upd_proj_body,
    out_shape=[jax.ShapeDtypeStruct((_N, 64), jnp.float32),
               jax.ShapeDtypeStruct((_N, 128), jnp.float32),
               jax.ShapeDtypeStruct((_N, 128), jnp.float32)],
)

_upd_final_call = pl.pallas_call(
    _upd_final_body,
    out_shape=jax.ShapeDtypeStruct((1, 1), jnp.float32),
)


def kernel(edge_index, h, e, lh, W_emb, b_emb, W_r1, b_r1, W_r2, b_r2,
           Wsrc, bsrc, Wdst, bdst, Wedge, bedge,
           bn_m_g, bn_m_b, bn_g, bn_b, W_f1, b_f1, W_f2, b_f2):
    src = edge_index[0]
    dst = edge_index[1]
    dst_half = [dst[:_E // 2], dst[_E // 2:]]
    dpar = (dst & 1).astype(jnp.float32).reshape(_NB, 1, _BE)

    x, hs, hd = _embed_proj_call(h, W_emb, b_emb.reshape(1, -1),
                                 Wsrc[0], bsrc[0].reshape(1, -1),
                                 Wdst[0], bdst[0].reshape(1, -1))
    ef = _ef_call(lh.reshape(_NB, 1, _BE), W_r1, b_r1.reshape(-1, 1),
                  W_r2, b_r2.reshape(1, -1))

    for l in range(3):
        g = _sc_gather_add(src, dst, hs, hd)
        st = _stats_call(g, ef, Wedge[l], bedge[l].reshape(1, -1))
        parts = []
        for hh in range(2):
            msg_h = _emit_half[hh](g, ef, Wedge[l], bedge[l].reshape(1, -1),
                                   st, bn_m_g[l].reshape(1, -1),
                                   bn_m_b[l].reshape(1, -1), dpar)
            parts.append(_sc_scatter_half(dst_half[hh],
                                          msg_h).reshape(_NC, _HN, 64))
        if l < 2:
            x, hs, hd = _upd_proj_call(parts[0], parts[1],
                                       x, bn_g[l].reshape(1, -1),
                                       bn_b[l].reshape(1, -1),
                                       Wsrc[l + 1], bsrc[l + 1].reshape(1, -1),
                                       Wdst[l + 1], bdst[l + 1].reshape(1, -1))
        else:
            out = _upd_final_call(parts[0], parts[1],
                                  x, bn_g[l].reshape(1, -1),
                                  bn_b[l].reshape(1, -1),
                                  W_f1, b_f1.reshape(1, -1),
                                  W_f2, b_f2.reshape(1, -1))

    return out.reshape(1)
